# Initial kernel scaffold; baseline (speedup 1.0000x reference)
import jax
import jax.numpy as jnp
from jax.experimental import pallas as pl


def kernel(x, edge_index, edge_attr, W_em, b_em, W0, b0, Wl1, bl1, Wr1, br1, We1, att1, bias1, Wl2, bl2, Wr2, br2, We2, att2, bias2, Wd, bd):
    raise NotImplementedError("write your pallas kernel here")


# speedup vs baseline: 13.8615x; 13.8615x over previous
import functools

import jax
import jax.numpy as jnp
from jax import lax
from jax.experimental import pallas as pl
from jax.experimental.pallas import tpu as pltpu
from jax.experimental.pallas import tpu_sc as plsc

NC = 2
NS = 16
NW = NC * NS
SUB = 80


def _mesh():
    return plsc.VectorSubcoreMesh(core_axis_name="c", subcore_axis_name="s",
                                  num_cores=NC, num_subcores=NS)


def _wid():
    return lax.axis_index("s") * NC + lax.axis_index("c")


def _make_g1(n, e):
    per_w = e // NW
    steps = per_w // 16

    @functools.partial(
        pl.kernel,
        out_type=[jax.ShapeDtypeStruct((e,), jnp.float32),
                  jax.ShapeDtypeStruct((e,), jnp.float32)],
        mesh=_mesh(),
        compiler_params=pltpu.CompilerParams(needs_layout_passes=False),
        scratch_types=[pltpu.VMEM((n,), jnp.float32),
                       pltpu.VMEM((per_w,), jnp.int32),
                       pltpu.VMEM((per_w,), jnp.float32)],
    )
    def g1(x_hbm, src_hbm, dst_hbm, xs_hbm, xd_hbm, x_v, idx_v, val_v):
        base = _wid() * per_w
        pltpu.sync_copy(x_hbm, x_v)
        for idx_hbm, out_hbm in ((src_hbm, xs_hbm), (dst_hbm, xd_hbm)):
            pltpu.sync_copy(idx_hbm.at[pl.ds(base, per_w)], idx_v)

            def body(i, carry):
                iv = idx_v[pl.ds(i * 16, 16)]
                val_v[pl.ds(i * 16, 16)] = plsc.load_gather(x_v, [iv])
                return carry

            lax.fori_loop(0, steps, body, 0)
            pltpu.sync_copy(val_v, out_hbm.at[pl.ds(base, per_w)])

    return g1


def _make_scatter(n, e, C):
    per_w = e // NW
    steps = per_w // SUB

    @functools.partial(
        pl.kernel,
        out_type=jax.ShapeDtypeStruct((NC, n, C), jnp.float32),
        mesh=_mesh(),
        compiler_params=pltpu.CompilerParams(needs_layout_passes=False),
        scratch_types=[pltpu.VMEM((SUB,), jnp.int32),
                       pltpu.VMEM((SUB, C), jnp.float32),
                       pltpu.VMEM_SHARED((n, C), jnp.float32)],
    )
    def scat(dst3_hbm, val4_hbm, zfull_hbm, out_hbm, idx_v, val_v, acc_sh):
        cid = lax.axis_index("c")
        sid = lax.axis_index("s")
        wid = sid * NC + cid

        @pl.when(sid == 0)
        def _():
            pltpu.sync_copy(zfull_hbm, acc_sh)

        plsc.subcore_barrier()

        def body(j, carry):
            pltpu.sync_copy(dst3_hbm.at[wid, j], idx_v)
            pltpu.sync_copy(val4_hbm.at[wid, j], val_v)
            pltpu.sync_copy(val_v, acc_sh.at[idx_v], add=True)
            return carry

        lax.fori_loop(0, steps, body, 0)
        plsc.subcore_barrier()

        @pl.when(sid == 0)
        def _():
            pltpu.sync_copy(acc_sh, out_hbm.at[cid])

    return scat


def _make_g2(n, e, D):
    per_w = e // NW
    steps = per_w // SUB

    @functools.partial(
        pl.kernel,
        out_type=[jax.ShapeDtypeStruct((e, D), jnp.float32),
                  jax.ShapeDtypeStruct((e, D), jnp.float32)],
        mesh=_mesh(),
        compiler_params=pltpu.CompilerParams(needs_layout_passes=False),
        scratch_types=[pltpu.VMEM((SUB,), jnp.int32),
                       pltpu.VMEM((SUB,), jnp.int32),
                       pltpu.VMEM((SUB, D), jnp.float32),
                       pltpu.VMEM((SUB, D), jnp.float32),
                       pltpu.SemaphoreType.DMA,
                       pltpu.SemaphoreType.DMA],
    )
    def g2(tl_hbm, tr_hbm, src3_hbm, dst3_hbm, gl_hbm, gr_hbm,
           si_v, di_v, row_l, row_r, sem_l, sem_r):
        wid = _wid()
        base = wid * per_w

        def body(j, carry):
            pltpu.sync_copy(src3_hbm.at[wid, j], si_v)
            pltpu.sync_copy(dst3_hbm.at[wid, j], di_v)
            cl = pltpu.async_copy(tl_hbm.at[si_v], row_l, sem_l)
            cr = pltpu.async_copy(tr_hbm.at[di_v], row_r, sem_r)
            cl.wait()
            cr.wait()
            off = base + j * SUB
            pltpu.sync_copy(row_l, gl_hbm.at[pl.ds(off, SUB)])
            pltpu.sync_copy(row_r, gr_hbm.at[pl.ds(off, SUB)])
            return carry

        lax.fori_loop(0, steps, body, 0)

    return g2


def _make_g3(n, e):
    per_w = e // NW
    steps = per_w // 16

    @functools.partial(
        pl.kernel,
        out_type=[jax.ShapeDtypeStruct((e,), jnp.float32),
                  jax.ShapeDtypeStruct((e,), jnp.float32)],
        mesh=_mesh(),
        compiler_params=pltpu.CompilerParams(needs_layout_passes=False),
        scratch_types=[pltpu.VMEM((n,), jnp.float32),
                       pltpu.VMEM((n,), jnp.float32),
                       pltpu.VMEM((per_w,), jnp.int32),
                       pltpu.VMEM((per_w,), jnp.float32),
                       pltpu.VMEM((per_w,), jnp.float32)],
    )
    def g3(lp_hbm, lq_hbm, src_hbm, lps_hbm, lqs_hbm, lp_v, lq_v, idx_v,
           vp_v, vq_v):
        base = _wid() * per_w
        pltpu.sync_copy(lp_hbm, lp_v)
        pltpu.sync_copy(lq_hbm, lq_v)
        pltpu.sync_copy(src_hbm.at[pl.ds(base, per_w)], idx_v)

        def body(i, carry):
            iv = idx_v[pl.ds(i * 16, 16)]
            vp_v[pl.ds(i * 16, 16)] = plsc.load_gather(lp_v, [iv])
            vq_v[pl.ds(i * 16, 16)] = plsc.load_gather(lq_v, [iv])
            return carry

        lax.fori_loop(0, steps, body, 0)
        pltpu.sync_copy(vp_v, lps_hbm.at[pl.ds(base, per_w)])
        pltpu.sync_copy(vq_v, lqs_hbm.at[pl.ds(base, per_w)])

    return g3


def _make_final(n, e):
    per_w = e // NW
    steps = per_w // 16

    @functools.partial(
        pl.kernel,
        out_type=jax.ShapeDtypeStruct((e,), jnp.float32),
        mesh=_mesh(),
        compiler_params=pltpu.CompilerParams(needs_layout_passes=False),
        scratch_types=[pltpu.VMEM((n,), jnp.float32),
                       pltpu.VMEM((n,), jnp.float32),
                       pltpu.VMEM((per_w,), jnp.int32),
                       pltpu.VMEM((per_w,), jnp.int32),
                       pltpu.VMEM((per_w,), jnp.float32)],
    )
    def fin(p_hbm, q_hbm, src_hbm, dst_hbm, out_hbm, p_v, q_v, si_v, di_v,
            val_v):
        base = _wid() * per_w
        pltpu.sync_copy(p_hbm, p_v)
        pltpu.sync_copy(q_hbm, q_v)
        pltpu.sync_copy(src_hbm.at[pl.ds(base, per_w)], si_v)
        pltpu.sync_copy(dst_hbm.at[pl.ds(base, per_w)], di_v)

        def body(i, carry):
            s16 = si_v[pl.ds(i * 16, 16)]
            d16 = di_v[pl.ds(i * 16, 16)]
            vp = plsc.load_gather(p_v, [s16])
            vq = plsc.load_gather(q_v, [d16])
            val_v[pl.ds(i * 16, 16)] = vp + vq
            return carry

        lax.fori_loop(0, steps, body, 0)
        pltpu.sync_copy(val_v, out_hbm.at[pl.ds(base, per_w)])

    return fin


def _e1_body(xs_ref, xd_ref, ea_ref, al_ref, ar_ref, v1_ref, k1_ref, m_ref,
             o_ref):
    xs = xs_ref[...]
    z = (xs * al_ref[...] + xd_ref[...] * ar_ref[...]
         + ea_ref[...] * v1_ref[...] + k1_ref[...])
    z = jnp.where(z > 0, z, 0.2 * z)
    logits = jnp.dot(z, m_ref[...], preferred_element_type=jnp.float32, precision=lax.Precision.HIGHEST)
    ex = jnp.exp(logits)
    o_ref[...] = jnp.concatenate([ex, xs * ex], axis=1)


def _n1_body(acc_ref, a_ref, b_ref, bias_ref, wl_ref, bl_ref, wr_ref, br_ref,
             wd_ref, xl_ref, xr_ref, lw_ref):
    a = acc_ref[0] + acc_ref[1]
    ssum = a[:, :8]
    sxe = a[:, 8:]
    den = ssum + 1e-16
    s0 = ssum / den
    s1 = sxe / den
    out1 = (jnp.dot(s1, a_ref[...], preferred_element_type=jnp.float32, precision=lax.Precision.HIGHEST)
            + jnp.dot(s0, b_ref[...], preferred_element_type=jnp.float32, precision=lax.Precision.HIGHEST)
            + bias_ref[...])
    h2 = jnp.where(out1 > 0, out1, jnp.exp(jnp.minimum(out1, 0.0)) - 1.0)
    xl = jnp.dot(h2, wl_ref[...],
                 preferred_element_type=jnp.float32, precision=lax.Precision.HIGHEST) + bl_ref[...]
    xl_ref[...] = xl
    xr_ref[...] = jnp.dot(h2, wr_ref[...],
                          preferred_element_type=jnp.float32, precision=lax.Precision.HIGHEST) + br_ref[...]
    lw_ref[...] = jnp.dot(xl, wd_ref[...],
                          preferred_element_type=jnp.float32, precision=lax.Precision.HIGHEST)


def _e2_body(gl_ref, gr_ref, ea_ref, lps_ref, lqs_ref, v2_ref, c2_ref,
             att_ref, o_ref):
    z = (gl_ref[...] + gr_ref[...] + ea_ref[...] * v2_ref[...]
         + c2_ref[...])
    z = jnp.where(z > 0, z, 0.2 * z)
    logit = jnp.dot(z, att_ref[...], preferred_element_type=jnp.float32, precision=lax.Precision.HIGHEST)
    ex = jnp.exp(logit)
    pad = jnp.zeros((ex.shape[0], 13), jnp.float32)
    o_ref[...] = jnp.concatenate(
        [ex * lps_ref[...], ex * lqs_ref[...], ex, pad], axis=1)


def _n2_body(acc_ref, cpq_ref, pq_ref):
    a = acc_ref[0] + acc_ref[1]
    den = a[:, 2:3] + 1e-16
    pq_ref[...] = a[:, 0:2] / den + cpq_ref[...]


def _row_spec(b, c):
    return pl.BlockSpec((b, c), lambda i: (i, 0))


def _full_spec(shape):
    return pl.BlockSpec(shape, lambda i: tuple(0 for _ in shape))


def kernel(x, edge_index, edge_attr, W_em, b_em, W0, b0, Wl1, bl1, Wr1, br1,
           We1, att1, bias1, Wl2, bl2, Wr2, br2, We2, att2, bias2, Wd, bd):
    n = x.shape[0]
    e = edge_index.shape[1]
    npsub = NS * SUB
    n_pad = ((n + npsub - 1) // npsub) * npsub
    src = edge_index[0]
    dst = edge_index[1]
    per_w = e // NW
    steps = per_w // SUB

    w0 = W0[:, 0]
    wem = W_em[:, 0]
    hi = functools.partial(jnp.dot, precision=lax.Precision.HIGHEST)
    al = hi(Wl1, w0)
    abl = hi(Wl1, b0) + bl1
    ar = hi(Wr1, w0)
    abr = hi(Wr1, b0) + br1
    v1 = hi(We1, wem)
    c1 = hi(We1, b_em)
    k1 = abl + abr + c1
    att1f = att1[0].reshape(64)
    hsel = (jnp.arange(64) // 8)[:, None] == jnp.arange(8)[None, :]
    m1 = jnp.where(hsel, att1f[:, None], 0.0)
    a_full = jnp.where(hsel.T, al[None, :], 0.0)
    b_full = jnp.where(hsel.T, abl[None, :], 0.0)
    v2 = hi(We2, wem)
    c2 = hi(We2, b_em)
    wd2 = jnp.stack([Wd[0, :128], Wd[0, 128:]], axis=1)
    cpq = jnp.stack([hi(bias2, wd2[:, 0]) + bd[0],
                     hi(bias2, wd2[:, 1])])[None]

    src3 = src.reshape(NW, steps, SUB)
    dst3 = dst.reshape(NW, steps, SUB)

    xs, xd = _make_g1(n, e)(x[:, 0], src, dst)

    BE = 512
    val1 = pl.pallas_call(
        _e1_body,
        grid=(e // BE,),
        in_specs=[_row_spec(BE, 1), _row_spec(BE, 1), _row_spec(BE, 1),
                  _full_spec((1, 64)), _full_spec((1, 64)),
                  _full_spec((1, 64)), _full_spec((1, 64)),
                  _full_spec((64, 8))],
        out_specs=_row_spec(BE, 16),
        out_shape=jax.ShapeDtypeStruct((e, 16), jnp.float32),
    )(xs[:, None], xd[:, None], edge_attr, al[None], ar[None], v1[None],
      k1[None], m1)

    zrow = jnp.zeros((n_pad, 16), jnp.float32)
    acc1 = jax.ops.segment_sum(val1, dst, num_segments=n_pad)[None]
    acc1 = jnp.concatenate([acc1, jnp.zeros_like(acc1)], axis=0)

    BN = 2048 if n_pad % 2048 == 0 else NS * SUB
    xl2, xr2, lw = pl.pallas_call(
        _n1_body,
        grid=(n_pad // BN,),
        in_specs=[pl.BlockSpec((NC, BN, 16), lambda i: (0, i, 0)),
                  _full_spec((8, 64)), _full_spec((8, 64)),
                  _full_spec((1, 64)), _full_spec((64, 128)),
                  _full_spec((1, 128)), _full_spec((64, 128)),
                  _full_spec((1, 128)), _full_spec((128, 2))],
        out_specs=[_row_spec(BN, 128), _row_spec(BN, 128), _row_spec(BN, 2)],
        out_shape=[jax.ShapeDtypeStruct((n_pad, 128), jnp.float32),
                   jax.ShapeDtypeStruct((n_pad, 128), jnp.float32),
                   jax.ShapeDtypeStruct((n_pad, 2), jnp.float32)],
    )(acc1, a_full, b_full, bias1[None], Wl2.T, bl2[None], Wr2.T, br2[None],
      wd2)

    gl, gr = _make_g2(n_pad, e, 128)(xl2, xr2, src3, dst3)

    lps, lqs = _make_g3(n_pad, e)(lw[:, 0], lw[:, 1], src)

    BE2 = 1000
    val2 = pl.pallas_call(
        _e2_body,
        grid=(e // BE2,),
        in_specs=[_row_spec(BE2, 128), _row_spec(BE2, 128), _row_spec(BE2, 1),
                  _row_spec(BE2, 1), _row_spec(BE2, 1),
                  _full_spec((1, 128)), _full_spec((1, 128)),
                  _full_spec((128, 1))],
        out_specs=_row_spec(BE2, 16),
        out_shape=jax.ShapeDtypeStruct((e, 16), jnp.float32),
    )(gl, gr, edge_attr, lps[:, None], lqs[:, None], v2[None], c2[None],
      att2[0].reshape(128, 1))

    acc2 = jax.ops.segment_sum(val2, dst, num_segments=n_pad)[None]
    acc2 = jnp.concatenate([acc2, jnp.zeros_like(acc2)], axis=0)

    pq = pl.pallas_call(
        _n2_body,
        grid=(n_pad // BN,),
        in_specs=[pl.BlockSpec((NC, BN, 16), lambda i: (0, i, 0)),
                  _full_spec((1, 2))],
        out_specs=_row_spec(BN, 2),
        out_shape=jax.ShapeDtypeStruct((n_pad, 2), jnp.float32),
    )(acc2, cpq)

    out = _make_final(n_pad, e)(pq[:, 0], pq[:, 1], src, dst)
    return out[:, None]
